# Initial kernel scaffold; baseline (speedup 1.0000x reference)
import jax
import jax.numpy as jnp
from jax.experimental import pallas as pl


def kernel(x, edge_index, edge_weight, batch, W1l, W1r, b1, W2l, W2r, b2, W3l, W3r, b3, L1W, L1b, L2W, L2b):
    raise NotImplementedError("write your pallas kernel here")


# speedup vs baseline: 6.1819x; 6.1819x over previous
import functools

import jax
import jax.numpy as jnp
from jax import lax
from jax.experimental import pallas as pl
from jax.experimental.pallas import tpu as pltpu
from jax.experimental.pallas import tpu_sc as plsc

N = 10000
E = 320000
DIN = 128
DH = 32
DOUT = 8
NG = 32

NC = 2
NS = 16
NW = NC * NS
EPT = E // NW
CH = 80
ITERS = EPT // CH
NP = 10240
RPT = NP // NS

BLK = 400
GRID = N // BLK


@functools.lru_cache(maxsize=None)
def _make_sc_aggregate(W):
  mesh = plsc.VectorSubcoreMesh(core_axis_name="c", subcore_axis_name="s",
                                num_cores=NC, num_subcores=NS)

  @functools.partial(
      pl.kernel,
      out_type=jax.ShapeDtypeStruct((NC, NP, W), jnp.float32),
      mesh=mesh,
      scratch_types=[
          pltpu.VMEM((CH,), jnp.int32),
          pltpu.VMEM((CH,), jnp.int32),
          pltpu.VMEM((CH, W), jnp.float32),
          pltpu.VMEM_SHARED((NP, W), jnp.float32),
          pltpu.SemaphoreType.DMA,
      ],
      compiler_params=pltpu.CompilerParams(use_tc_tiling_on_sc=False),
  )
  def agg(p_hbm, src_hbm, dst_hbm, zeros_hbm, out_hbm,
          sidx, didx, rows, acc, sem):
    c = lax.axis_index("c")
    s = lax.axis_index("s")
    wid = c * NS + s
    pltpu.sync_copy(zeros_hbm.at[pl.ds(s * RPT, RPT)],
                    acc.at[pl.ds(s * RPT, RPT)])
    plsc.subcore_barrier()
    base = wid * EPT

    def body(i, carry):
      off = base + i * CH
      pltpu.sync_copy(src_hbm.at[pl.ds(off, CH)], sidx)
      pltpu.sync_copy(dst_hbm.at[pl.ds(off, CH)], didx)
      pltpu.async_copy(p_hbm.at[sidx], rows, sem).wait()
      pltpu.sync_copy(rows, acc.at[didx], add=True)
      return carry

    lax.fori_loop(0, ITERS, body, 0)
    plsc.subcore_barrier()
    pltpu.sync_copy(acc.at[pl.ds(s * RPT, RPT)],
                    out_hbm.at[c, pl.ds(s * RPT, RPT)])

  return agg


def _k0_body(x_ref, wl_ref, wr_ref, b_ref, paug_ref, q_ref):
  x = x_ref[...]
  p = jnp.dot(x, wl_ref[...], preferred_element_type=jnp.float32)
  one = jnp.ones((BLK, 1), jnp.float32)
  pad = jnp.zeros((BLK, 15), jnp.float32)
  paug_ref[...] = jnp.concatenate([p, one, pad], axis=1)
  q_ref[...] = jnp.dot(x, wr_ref[...], preferred_element_type=jnp.float32) + b_ref[...]


def _tc_project_in(x, Wl, Wr, b):
  return pl.pallas_call(
      _k0_body,
      grid=(GRID,),
      in_specs=[
          pl.BlockSpec((BLK, DIN), lambda i: (i, 0)),
          pl.BlockSpec((DIN, DH), lambda i: (0, 0)),
          pl.BlockSpec((DIN, DH), lambda i: (0, 0)),
          pl.BlockSpec((1, DH), lambda i: (0, 0)),
      ],
      out_specs=[
          pl.BlockSpec((BLK, DH + 16), lambda i: (i, 0)),
          pl.BlockSpec((BLK, DH), lambda i: (i, 0)),
      ],
      out_shape=[
          jax.ShapeDtypeStruct((N, DH + 16), jnp.float32),
          jax.ShapeDtypeStruct((N, DH), jnp.float32),
      ],
  )(x, Wl, Wr, b.reshape(1, DH))


def _k1_body(a0_ref, a1_ref, q_ref, wl_ref, wr_ref, b_ref,
             p_ref, qn_ref, cnt_ref):
  a0 = a0_ref[...]
  a1 = a1_ref[...]
  cnt = a0[:, DH:DH + 1] + a1[:, DH:DH + 1]
  agg = a0[:, :DH] + a1[:, :DH]
  h = jnp.maximum(agg / jnp.maximum(cnt, 1.0) + q_ref[...], 0.0)
  p_ref[...] = jnp.dot(h, wl_ref[...], preferred_element_type=jnp.float32)
  qn_ref[...] = jnp.dot(h, wr_ref[...], preferred_element_type=jnp.float32) + b_ref[...]
  cnt_ref[...] = cnt


def _tc_combine1(a0, a1, q, Wl, Wr, b):
  return pl.pallas_call(
      _k1_body,
      grid=(GRID,),
      in_specs=[
          pl.BlockSpec((BLK, DH + 16), lambda i: (i, 0)),
          pl.BlockSpec((BLK, DH + 16), lambda i: (i, 0)),
          pl.BlockSpec((BLK, DH), lambda i: (i, 0)),
          pl.BlockSpec((DH, DH), lambda i: (0, 0)),
          pl.BlockSpec((DH, DH), lambda i: (0, 0)),
          pl.BlockSpec((1, DH), lambda i: (0, 0)),
      ],
      out_specs=[
          pl.BlockSpec((BLK, DH), lambda i: (i, 0)),
          pl.BlockSpec((BLK, DH), lambda i: (i, 0)),
          pl.BlockSpec((BLK, 1), lambda i: (i, 0)),
      ],
      out_shape=[
          jax.ShapeDtypeStruct((N, DH), jnp.float32),
          jax.ShapeDtypeStruct((N, DH), jnp.float32),
          jax.ShapeDtypeStruct((N, 1), jnp.float32),
      ],
  )(a0, a1, q, Wl, Wr, b.reshape(1, DH))


def _k2_body(a0_ref, a1_ref, q_ref, cnt_ref, wl_ref, wr_ref, b_ref,
             p_ref, qn_ref):
  agg = a0_ref[...] + a1_ref[...]
  h = jnp.maximum(agg / jnp.maximum(cnt_ref[...], 1.0) + q_ref[...], 0.0)
  p_ref[...] = jnp.dot(h, wl_ref[...], preferred_element_type=jnp.float32)
  qn_ref[...] = jnp.dot(h, wr_ref[...], preferred_element_type=jnp.float32) + b_ref[...]


def _tc_combine2(a0, a1, q, cnt, Wl, Wr, b):
  return pl.pallas_call(
      _k2_body,
      grid=(GRID,),
      in_specs=[
          pl.BlockSpec((BLK, DH), lambda i: (i, 0)),
          pl.BlockSpec((BLK, DH), lambda i: (i, 0)),
          pl.BlockSpec((BLK, DH), lambda i: (i, 0)),
          pl.BlockSpec((BLK, 1), lambda i: (i, 0)),
          pl.BlockSpec((DH, DH), lambda i: (0, 0)),
          pl.BlockSpec((DH, DH), lambda i: (0, 0)),
          pl.BlockSpec((1, DH), lambda i: (0, 0)),
      ],
      out_specs=[
          pl.BlockSpec((BLK, DH), lambda i: (i, 0)),
          pl.BlockSpec((BLK, DH), lambda i: (i, 0)),
      ],
      out_shape=[
          jax.ShapeDtypeStruct((N, DH), jnp.float32),
          jax.ShapeDtypeStruct((N, DH), jnp.float32),
      ],
  )(a0, a1, q, cnt, Wl, Wr, b.reshape(1, DH))


def _k3_body(a0_ref, a1_ref, q_ref, cnt_ref, batch_ref,
             l1w_ref, l1b_ref, l2w_ref, l2b_ref, out_ref,
             gs_ref, gc_ref):
  i = pl.program_id(0)

  @pl.when(i == 0)
  def _():
    gs_ref[...] = jnp.zeros_like(gs_ref)
    gc_ref[...] = jnp.zeros_like(gc_ref)

  agg = a0_ref[...] + a1_ref[...]
  h = jnp.maximum(agg / jnp.maximum(cnt_ref[...], 1.0) + q_ref[...], 0.0)
  groups = lax.broadcasted_iota(jnp.int32, (BLK, NG), 1)
  onehot = (batch_ref[...] == groups).astype(jnp.float32)
  gs_ref[...] += lax.dot_general(
      onehot, h, (((0,), (0,)), ((), ())),
      preferred_element_type=jnp.float32)
  gc_ref[...] += lax.dot_general(
      onehot, jnp.ones((BLK, DH), jnp.float32), (((0,), (0,)), ((), ())),
      preferred_element_type=jnp.float32)

  @pl.when(i == GRID - 1)
  def _():
    g = gs_ref[...] / jnp.maximum(gc_ref[...], 1.0)
    z = jnp.maximum(
        jnp.dot(g, l1w_ref[...], preferred_element_type=jnp.float32)
        + l1b_ref[...], 0.0)
    out_ref[...] = jnp.dot(
        z, l2w_ref[...], preferred_element_type=jnp.float32) + l2b_ref[...]


def _tc_head(a0, a1, q, cnt, batch2d, L1W, L1b, L2W, L2b):
  return pl.pallas_call(
      _k3_body,
      grid=(GRID,),
      in_specs=[
          pl.BlockSpec((BLK, DH), lambda i: (i, 0)),
          pl.BlockSpec((BLK, DH), lambda i: (i, 0)),
          pl.BlockSpec((BLK, DH), lambda i: (i, 0)),
          pl.BlockSpec((BLK, 1), lambda i: (i, 0)),
          pl.BlockSpec((BLK, 1), lambda i: (i, 0)),
          pl.BlockSpec((DH, DH), lambda i: (0, 0)),
          pl.BlockSpec((1, DH), lambda i: (0, 0)),
          pl.BlockSpec((DH, DOUT), lambda i: (0, 0)),
          pl.BlockSpec((1, DOUT), lambda i: (0, 0)),
      ],
      out_specs=pl.BlockSpec((NG, DOUT), lambda i: (0, 0)),
      out_shape=jax.ShapeDtypeStruct((NG, DOUT), jnp.float32),
      scratch_shapes=[
          pltpu.VMEM((NG, DH), jnp.float32),
          pltpu.VMEM((NG, DH), jnp.float32),
      ],
  )(a0, a1, q, cnt, batch2d, L1W, L1b.reshape(1, DH), L2W, L2b.reshape(1, DOUT))


def kernel(x, edge_index, edge_weight, batch,
           W1l, W1r, b1, W2l, W2r, b2, W3l, W3r, b3,
           L1W, L1b, L2W, L2b):
  del edge_weight
  src = edge_index[0]
  dst = edge_index[1]
  zeros48 = jnp.zeros((NP, DH + 16), jnp.float32)
  zeros32 = jnp.zeros((NP, DH), jnp.float32)

  p1, q1 = _tc_project_in(x, W1l, W1r, b1)
  acc1 = _make_sc_aggregate(DH + 16)(p1, src, dst, zeros48)
  p2, q2, cnt = _tc_combine1(acc1[0], acc1[1], q1, W2l, W2r, b2)
  acc2 = _make_sc_aggregate(DH)(p2, src, dst, zeros32)
  p3, q3 = _tc_combine2(acc2[0], acc2[1], q2, cnt, W3l, W3r, b3)
  acc3 = _make_sc_aggregate(DH)(p3, src, dst, zeros32)
  out = _tc_head(acc3[0], acc3[1], q3, cnt, batch.reshape(N, 1),
                 L1W, L1b, L2W, L2b)
  return out


# speedup vs baseline: 7.8233x; 1.2655x over previous
import functools

import jax
import jax.numpy as jnp
from jax import lax
from jax.experimental import pallas as pl
from jax.experimental.pallas import tpu as pltpu
from jax.experimental.pallas import tpu_sc as plsc

N = 10000
E = 320000
DIN = 128
DH = 32
DOUT = 8
NG = 32

NC = 2
NS = 16
NW = NC * NS
CH = 128
EP = 327680
EPT = EP // NW
ITERS = EPT // CH
NP = 10240
RPT = NP // NS

BLK = 400
GRID = N // BLK


@functools.lru_cache(maxsize=None)
def _make_sc_aggregate(W):
  mesh = plsc.VectorSubcoreMesh(core_axis_name="c", subcore_axis_name="s",
                                num_cores=NC, num_subcores=NS)

  @functools.partial(
      pl.kernel,
      out_type=jax.ShapeDtypeStruct((NC, NP, W), jnp.float32),
      mesh=mesh,
      scratch_types=[
          pltpu.VMEM((ITERS, CH), jnp.int32),
          pltpu.VMEM((ITERS, CH), jnp.int32),
          pltpu.VMEM((CH, W), jnp.float32),
          pltpu.VMEM((CH, W), jnp.float32),
          pltpu.VMEM_SHARED((NP, W), jnp.float32),
          pltpu.SemaphoreType.DMA,
          pltpu.SemaphoreType.DMA,
      ],
      compiler_params=pltpu.CompilerParams(use_tc_tiling_on_sc=False),
  )
  def agg(p_hbm, src_hbm, dst_hbm, zeros_hbm, out_hbm,
          sidx, didx, rows0, rows1, acc, gsem, ssem):
    c = lax.axis_index("c")
    s = lax.axis_index("s")
    wid = c * NS + s
    pltpu.sync_copy(zeros_hbm.at[pl.ds(s * RPT, RPT)],
                    acc.at[pl.ds(s * RPT, RPT)])
    row0 = wid * ITERS
    pltpu.sync_copy(src_hbm.at[pl.ds(row0, ITERS)], sidx)
    pltpu.sync_copy(dst_hbm.at[pl.ds(row0, ITERS)], didx)
    plsc.subcore_barrier()

    def start_gather(j, buf):
      pltpu.async_copy(p_hbm.at[sidx.at[j]], buf, gsem)

    def start_scatter(j, buf):
      pltpu.async_copy(buf, acc.at[didx.at[j]], ssem, add=True)

    def wait_gather(buf):
      pltpu.make_async_copy(p_hbm.at[sidx.at[0]], buf, gsem).wait()

    def wait_scatter(buf):
      pltpu.make_async_copy(buf, acc.at[didx.at[0]], ssem).wait()

    start_gather(0, rows0)

    def body(j2, carry):
      a = 2 * j2
      wait_gather(rows0)

      @pl.when(j2 > 0)
      def _():
        wait_scatter(rows1)

      start_gather(a + 1, rows1)
      start_scatter(a, rows0)
      wait_gather(rows1)
      wait_scatter(rows0)

      @pl.when(j2 < ITERS // 2 - 1)
      def _():
        start_gather(a + 2, rows0)

      start_scatter(a + 1, rows1)
      return carry

    lax.fori_loop(0, ITERS // 2, body, 0)
    wait_scatter(rows1)
    plsc.subcore_barrier()
    pltpu.sync_copy(acc.at[pl.ds(s * RPT, RPT)],
                    out_hbm.at[c, pl.ds(s * RPT, RPT)])

  return agg


def _k0_body(x_ref, wl_ref, wr_ref, b_ref, paug_ref, q_ref):
  x = x_ref[...]
  p = jnp.dot(x, wl_ref[...], preferred_element_type=jnp.float32)
  one = jnp.ones((BLK, 1), jnp.float32)
  pad = jnp.zeros((BLK, 15), jnp.float32)
  paug_ref[...] = jnp.concatenate([p, one, pad], axis=1)
  q_ref[...] = jnp.dot(x, wr_ref[...], preferred_element_type=jnp.float32) + b_ref[...]


def _tc_project_in(x, Wl, Wr, b):
  return pl.pallas_call(
      _k0_body,
      grid=(GRID,),
      in_specs=[
          pl.BlockSpec((BLK, DIN), lambda i: (i, 0)),
          pl.BlockSpec((DIN, DH), lambda i: (0, 0)),
          pl.BlockSpec((DIN, DH), lambda i: (0, 0)),
          pl.BlockSpec((1, DH), lambda i: (0, 0)),
      ],
      out_specs=[
          pl.BlockSpec((BLK, DH + 16), lambda i: (i, 0)),
          pl.BlockSpec((BLK, DH), lambda i: (i, 0)),
      ],
      out_shape=[
          jax.ShapeDtypeStruct((N, DH + 16), jnp.float32),
          jax.ShapeDtypeStruct((N, DH), jnp.float32),
      ],
  )(x, Wl, Wr, b.reshape(1, DH))


def _k1_body(a0_ref, a1_ref, q_ref, wl_ref, wr_ref, b_ref,
             p_ref, qn_ref, cnt_ref):
  a0 = a0_ref[...]
  a1 = a1_ref[...]
  cnt = a0[:, DH:DH + 1] + a1[:, DH:DH + 1]
  agg = a0[:, :DH] + a1[:, :DH]
  h = jnp.maximum(agg / jnp.maximum(cnt, 1.0) + q_ref[...], 0.0)
  p_ref[...] = jnp.dot(h, wl_ref[...], preferred_element_type=jnp.float32)
  qn_ref[...] = jnp.dot(h, wr_ref[...], preferred_element_type=jnp.float32) + b_ref[...]
  cnt_ref[...] = cnt


def _tc_combine1(a0, a1, q, Wl, Wr, b):
  return pl.pallas_call(
      _k1_body,
      grid=(GRID,),
      in_specs=[
          pl.BlockSpec((BLK, DH + 16), lambda i: (i, 0)),
          pl.BlockSpec((BLK, DH + 16), lambda i: (i, 0)),
          pl.BlockSpec((BLK, DH), lambda i: (i, 0)),
          pl.BlockSpec((DH, DH), lambda i: (0, 0)),
          pl.BlockSpec((DH, DH), lambda i: (0, 0)),
          pl.BlockSpec((1, DH), lambda i: (0, 0)),
      ],
      out_specs=[
          pl.BlockSpec((BLK, DH), lambda i: (i, 0)),
          pl.BlockSpec((BLK, DH), lambda i: (i, 0)),
          pl.BlockSpec((BLK, 1), lambda i: (i, 0)),
      ],
      out_shape=[
          jax.ShapeDtypeStruct((N, DH), jnp.float32),
          jax.ShapeDtypeStruct((N, DH), jnp.float32),
          jax.ShapeDtypeStruct((N, 1), jnp.float32),
      ],
  )(a0, a1, q, Wl, Wr, b.reshape(1, DH))


def _k2_body(a0_ref, a1_ref, q_ref, cnt_ref, wl_ref, wr_ref, b_ref,
             p_ref, qn_ref):
  agg = a0_ref[...] + a1_ref[...]
  h = jnp.maximum(agg / jnp.maximum(cnt_ref[...], 1.0) + q_ref[...], 0.0)
  p_ref[...] = jnp.dot(h, wl_ref[...], preferred_element_type=jnp.float32)
  qn_ref[...] = jnp.dot(h, wr_ref[...], preferred_element_type=jnp.float32) + b_ref[...]


def _tc_combine2(a0, a1, q, cnt, Wl, Wr, b):
  return pl.pallas_call(
      _k2_body,
      grid=(GRID,),
      in_specs=[
          pl.BlockSpec((BLK, DH), lambda i: (i, 0)),
          pl.BlockSpec((BLK, DH), lambda i: (i, 0)),
          pl.BlockSpec((BLK, DH), lambda i: (i, 0)),
          pl.BlockSpec((BLK, 1), lambda i: (i, 0)),
          pl.BlockSpec((DH, DH), lambda i: (0, 0)),
          pl.BlockSpec((DH, DH), lambda i: (0, 0)),
          pl.BlockSpec((1, DH), lambda i: (0, 0)),
      ],
      out_specs=[
          pl.BlockSpec((BLK, DH), lambda i: (i, 0)),
          pl.BlockSpec((BLK, DH), lambda i: (i, 0)),
      ],
      out_shape=[
          jax.ShapeDtypeStruct((N, DH), jnp.float32),
          jax.ShapeDtypeStruct((N, DH), jnp.float32),
      ],
  )(a0, a1, q, cnt, Wl, Wr, b.reshape(1, DH))


def _k3_body(a0_ref, a1_ref, q_ref, cnt_ref, batch_ref,
             l1w_ref, l1b_ref, l2w_ref, l2b_ref, out_ref,
             gs_ref, gc_ref):
  i = pl.program_id(0)

  @pl.when(i == 0)
  def _():
    gs_ref[...] = jnp.zeros_like(gs_ref)
    gc_ref[...] = jnp.zeros_like(gc_ref)

  agg = a0_ref[...] + a1_ref[...]
  h = jnp.maximum(agg / jnp.maximum(cnt_ref[...], 1.0) + q_ref[...], 0.0)
  groups = lax.broadcasted_iota(jnp.int32, (BLK, NG), 1)
  onehot = (batch_ref[...] == groups).astype(jnp.float32)
  gs_ref[...] += lax.dot_general(
      onehot, h, (((0,), (0,)), ((), ())),
      preferred_element_type=jnp.float32)
  gc_ref[...] += lax.dot_general(
      onehot, jnp.ones((BLK, DH), jnp.float32), (((0,), (0,)), ((), ())),
      preferred_element_type=jnp.float32)

  @pl.when(i == GRID - 1)
  def _():
    g = gs_ref[...] / jnp.maximum(gc_ref[...], 1.0)
    z = jnp.maximum(
        jnp.dot(g, l1w_ref[...], preferred_element_type=jnp.float32)
        + l1b_ref[...], 0.0)
    out_ref[...] = jnp.dot(
        z, l2w_ref[...], preferred_element_type=jnp.float32) + l2b_ref[...]


def _tc_head(a0, a1, q, cnt, batch2d, L1W, L1b, L2W, L2b):
  return pl.pallas_call(
      _k3_body,
      grid=(GRID,),
      in_specs=[
          pl.BlockSpec((BLK, DH), lambda i: (i, 0)),
          pl.BlockSpec((BLK, DH), lambda i: (i, 0)),
          pl.BlockSpec((BLK, DH), lambda i: (i, 0)),
          pl.BlockSpec((BLK, 1), lambda i: (i, 0)),
          pl.BlockSpec((BLK, 1), lambda i: (i, 0)),
          pl.BlockSpec((DH, DH), lambda i: (0, 0)),
          pl.BlockSpec((1, DH), lambda i: (0, 0)),
          pl.BlockSpec((DH, DOUT), lambda i: (0, 0)),
          pl.BlockSpec((1, DOUT), lambda i: (0, 0)),
      ],
      out_specs=pl.BlockSpec((NG, DOUT), lambda i: (0, 0)),
      out_shape=jax.ShapeDtypeStruct((NG, DOUT), jnp.float32),
      scratch_shapes=[
          pltpu.VMEM((NG, DH), jnp.float32),
          pltpu.VMEM((NG, DH), jnp.float32),
      ],
  )(a0, a1, q, cnt, batch2d, L1W, L1b.reshape(1, DH), L2W, L2b.reshape(1, DOUT))


def kernel(x, edge_index, edge_weight, batch,
           W1l, W1r, b1, W2l, W2r, b2, W3l, W3r, b3,
           L1W, L1b, L2W, L2b):
  del edge_weight
  pad = EP - E
  src = jnp.concatenate(
      [edge_index[0], jnp.zeros((pad,), jnp.int32)]).reshape(EP // CH, CH)
  dst = jnp.concatenate(
      [edge_index[1], jnp.full((pad,), NP - 1, jnp.int32)]).reshape(EP // CH, CH)
  zeros48 = jnp.zeros((NP, DH + 16), jnp.float32)
  zeros32 = jnp.zeros((NP, DH), jnp.float32)

  p1, q1 = _tc_project_in(x, W1l, W1r, b1)
  acc1 = _make_sc_aggregate(DH + 16)(p1, src, dst, zeros48)
  p2, q2, cnt = _tc_combine1(acc1[0], acc1[1], q1, W2l, W2r, b2)
  acc2 = _make_sc_aggregate(DH)(p2, src, dst, zeros32)
  p3, q3 = _tc_combine2(acc2[0], acc2[1], q2, cnt, W3l, W3r, b3)
  acc3 = _make_sc_aggregate(DH)(p3, src, dst, zeros32)
  out = _tc_head(acc3[0], acc3[1], q3, cnt, batch.reshape(N, 1),
                 L1W, L1b, L2W, L2b)
  return out


# speedup vs baseline: 17.7220x; 2.2653x over previous
import functools

import jax
import jax.numpy as jnp
from jax import lax
from jax.experimental import pallas as pl
from jax.experimental.pallas import tpu as pltpu
from jax.experimental.pallas import tpu_sc as plsc

N = 10000
E = 320000
DIN = 128
DH = 32
DOUT = 8
NG = 32

NC = 2
NS = 16
NW = NC * NS
CH = 128
EP = 327680
EPT = EP // NW
ITERS = EPT // CH
NP = 10240
RPT = NP // NS

BLK = 2000
GRID = N // BLK


@functools.lru_cache(maxsize=None)
def _make_sc_aggregate(W):
  mesh = plsc.VectorSubcoreMesh(core_axis_name="c", subcore_axis_name="s",
                                num_cores=NC, num_subcores=NS)

  @functools.partial(
      pl.kernel,
      out_type=jax.ShapeDtypeStruct((NC, NP, W), jnp.float32),
      mesh=mesh,
      scratch_types=[
          pltpu.VMEM((ITERS, CH), jnp.int32),
          pltpu.VMEM((ITERS, CH), jnp.int32),
          pltpu.VMEM((CH, W), jnp.float32),
          pltpu.VMEM((CH, W), jnp.float32),
          pltpu.VMEM_SHARED((N, W), jnp.float32),
          pltpu.VMEM_SHARED((NP, W), jnp.float32),
          pltpu.SemaphoreType.DMA,
          pltpu.SemaphoreType.DMA,
      ],
      compiler_params=pltpu.CompilerParams(use_tc_tiling_on_sc=False),
  )
  def agg(p_hbm, src_hbm, dst_hbm, zeros_hbm, out_hbm,
          sidx, didx, rows0, rows1, p_sh, acc, gsem, ssem):
    c = lax.axis_index("c")
    s = lax.axis_index("s")
    wid = c * NS + s
    rps = N // NS
    pltpu.sync_copy(p_hbm.at[pl.ds(s * rps, rps)],
                    p_sh.at[pl.ds(s * rps, rps)])
    pltpu.sync_copy(zeros_hbm.at[pl.ds(s * RPT, RPT)],
                    acc.at[pl.ds(s * RPT, RPT)])
    row0 = wid * ITERS
    pltpu.sync_copy(src_hbm.at[pl.ds(row0, ITERS)], sidx)
    pltpu.sync_copy(dst_hbm.at[pl.ds(row0, ITERS)], didx)
    plsc.subcore_barrier()

    def start_gather(j, buf):
      pltpu.async_copy(p_sh.at[sidx.at[j]], buf, gsem)

    def start_scatter(j, buf):
      pltpu.async_copy(buf, acc.at[didx.at[j]], ssem, add=True)

    def wait_gather(buf):
      pltpu.make_async_copy(p_sh.at[sidx.at[0]], buf, gsem).wait()

    def wait_scatter(buf):
      pltpu.make_async_copy(buf, acc.at[didx.at[0]], ssem).wait()

    start_gather(0, rows0)

    def body(j2, carry):
      a = 2 * j2
      wait_gather(rows0)

      @pl.when(j2 > 0)
      def _():
        wait_scatter(rows1)

      start_gather(a + 1, rows1)
      start_scatter(a, rows0)
      wait_gather(rows1)
      wait_scatter(rows0)

      @pl.when(j2 < ITERS // 2 - 1)
      def _():
        start_gather(a + 2, rows0)

      start_scatter(a + 1, rows1)
      return carry

    lax.fori_loop(0, ITERS // 2, body, 0)
    wait_scatter(rows1)
    plsc.subcore_barrier()
    pltpu.sync_copy(acc.at[pl.ds(s * RPT, RPT)],
                    out_hbm.at[c, pl.ds(s * RPT, RPT)])

  return agg


def _k0_body(x_ref, wl_ref, wr_ref, b_ref, paug_ref, q_ref):
  x = x_ref[...]
  p = jnp.dot(x, wl_ref[...], preferred_element_type=jnp.float32)
  one = jnp.ones((BLK, 1), jnp.float32)
  pad = jnp.zeros((BLK, 15), jnp.float32)
  paug_ref[...] = jnp.concatenate([p, one, pad], axis=1)
  q_ref[...] = jnp.dot(x, wr_ref[...], preferred_element_type=jnp.float32) + b_ref[...]


def _tc_project_in(x, Wl, Wr, b):
  return pl.pallas_call(
      _k0_body,
      grid=(GRID,),
      in_specs=[
          pl.BlockSpec((BLK, DIN), lambda i: (i, 0)),
          pl.BlockSpec((DIN, DH), lambda i: (0, 0)),
          pl.BlockSpec((DIN, DH), lambda i: (0, 0)),
          pl.BlockSpec((1, DH), lambda i: (0, 0)),
      ],
      out_specs=[
          pl.BlockSpec((BLK, DH + 16), lambda i: (i, 0)),
          pl.BlockSpec((BLK, DH), lambda i: (i, 0)),
      ],
      out_shape=[
          jax.ShapeDtypeStruct((N, DH + 16), jnp.float32),
          jax.ShapeDtypeStruct((N, DH), jnp.float32),
      ],
  )(x, Wl, Wr, b.reshape(1, DH))


def _k1_body(a0_ref, a1_ref, q_ref, wl_ref, wr_ref, b_ref,
             p_ref, qn_ref, cnt_ref):
  a0 = a0_ref[...]
  a1 = a1_ref[...]
  cnt = a0[:, DH:DH + 1] + a1[:, DH:DH + 1]
  agg = a0[:, :DH] + a1[:, :DH]
  h = jnp.maximum(agg / jnp.maximum(cnt, 1.0) + q_ref[...], 0.0)
  p_ref[...] = jnp.dot(h, wl_ref[...], preferred_element_type=jnp.float32)
  qn_ref[...] = jnp.dot(h, wr_ref[...], preferred_element_type=jnp.float32) + b_ref[...]
  cnt_ref[...] = cnt


def _tc_combine1(a0, a1, q, Wl, Wr, b):
  return pl.pallas_call(
      _k1_body,
      grid=(GRID,),
      in_specs=[
          pl.BlockSpec((BLK, DH + 16), lambda i: (i, 0)),
          pl.BlockSpec((BLK, DH + 16), lambda i: (i, 0)),
          pl.BlockSpec((BLK, DH), lambda i: (i, 0)),
          pl.BlockSpec((DH, DH), lambda i: (0, 0)),
          pl.BlockSpec((DH, DH), lambda i: (0, 0)),
          pl.BlockSpec((1, DH), lambda i: (0, 0)),
      ],
      out_specs=[
          pl.BlockSpec((BLK, DH), lambda i: (i, 0)),
          pl.BlockSpec((BLK, DH), lambda i: (i, 0)),
          pl.BlockSpec((BLK, 1), lambda i: (i, 0)),
      ],
      out_shape=[
          jax.ShapeDtypeStruct((N, DH), jnp.float32),
          jax.ShapeDtypeStruct((N, DH), jnp.float32),
          jax.ShapeDtypeStruct((N, 1), jnp.float32),
      ],
  )(a0, a1, q, Wl, Wr, b.reshape(1, DH))


def _k2_body(a0_ref, a1_ref, q_ref, cnt_ref, wl_ref, wr_ref, b_ref,
             p_ref, qn_ref):
  agg = a0_ref[...] + a1_ref[...]
  h = jnp.maximum(agg / jnp.maximum(cnt_ref[...], 1.0) + q_ref[...], 0.0)
  p_ref[...] = jnp.dot(h, wl_ref[...], preferred_element_type=jnp.float32)
  qn_ref[...] = jnp.dot(h, wr_ref[...], preferred_element_type=jnp.float32) + b_ref[...]


def _tc_combine2(a0, a1, q, cnt, Wl, Wr, b):
  return pl.pallas_call(
      _k2_body,
      grid=(GRID,),
      in_specs=[
          pl.BlockSpec((BLK, DH), lambda i: (i, 0)),
          pl.BlockSpec((BLK, DH), lambda i: (i, 0)),
          pl.BlockSpec((BLK, DH), lambda i: (i, 0)),
          pl.BlockSpec((BLK, 1), lambda i: (i, 0)),
          pl.BlockSpec((DH, DH), lambda i: (0, 0)),
          pl.BlockSpec((DH, DH), lambda i: (0, 0)),
          pl.BlockSpec((1, DH), lambda i: (0, 0)),
      ],
      out_specs=[
          pl.BlockSpec((BLK, DH), lambda i: (i, 0)),
          pl.BlockSpec((BLK, DH), lambda i: (i, 0)),
      ],
      out_shape=[
          jax.ShapeDtypeStruct((N, DH), jnp.float32),
          jax.ShapeDtypeStruct((N, DH), jnp.float32),
      ],
  )(a0, a1, q, cnt, Wl, Wr, b.reshape(1, DH))


def _k3_body(a0_ref, a1_ref, q_ref, cnt_ref, batch_ref,
             l1w_ref, l1b_ref, l2w_ref, l2b_ref, out_ref,
             gs_ref, gc_ref):
  i = pl.program_id(0)

  @pl.when(i == 0)
  def _():
    gs_ref[...] = jnp.zeros_like(gs_ref)
    gc_ref[...] = jnp.zeros_like(gc_ref)

  agg = a0_ref[...] + a1_ref[...]
  h = jnp.maximum(agg / jnp.maximum(cnt_ref[...], 1.0) + q_ref[...], 0.0)
  groups = lax.broadcasted_iota(jnp.int32, (BLK, NG), 1)
  onehot = (batch_ref[...] == groups).astype(jnp.float32)
  gs_ref[...] += lax.dot_general(
      onehot, h, (((0,), (0,)), ((), ())),
      preferred_element_type=jnp.float32)
  gc_ref[...] += lax.dot_general(
      onehot, jnp.ones((BLK, DH), jnp.float32), (((0,), (0,)), ((), ())),
      preferred_element_type=jnp.float32)

  @pl.when(i == GRID - 1)
  def _():
    g = gs_ref[...] / jnp.maximum(gc_ref[...], 1.0)
    z = jnp.maximum(
        jnp.dot(g, l1w_ref[...], preferred_element_type=jnp.float32)
        + l1b_ref[...], 0.0)
    out_ref[...] = jnp.dot(
        z, l2w_ref[...], preferred_element_type=jnp.float32) + l2b_ref[...]


def _tc_head(a0, a1, q, cnt, batch2d, L1W, L1b, L2W, L2b):
  return pl.pallas_call(
      _k3_body,
      grid=(GRID,),
      in_specs=[
          pl.BlockSpec((BLK, DH), lambda i: (i, 0)),
          pl.BlockSpec((BLK, DH), lambda i: (i, 0)),
          pl.BlockSpec((BLK, DH), lambda i: (i, 0)),
          pl.BlockSpec((BLK, 1), lambda i: (i, 0)),
          pl.BlockSpec((BLK, 1), lambda i: (i, 0)),
          pl.BlockSpec((DH, DH), lambda i: (0, 0)),
          pl.BlockSpec((1, DH), lambda i: (0, 0)),
          pl.BlockSpec((DH, DOUT), lambda i: (0, 0)),
          pl.BlockSpec((1, DOUT), lambda i: (0, 0)),
      ],
      out_specs=pl.BlockSpec((NG, DOUT), lambda i: (0, 0)),
      out_shape=jax.ShapeDtypeStruct((NG, DOUT), jnp.float32),
      scratch_shapes=[
          pltpu.VMEM((NG, DH), jnp.float32),
          pltpu.VMEM((NG, DH), jnp.float32),
      ],
  )(a0, a1, q, cnt, batch2d, L1W, L1b.reshape(1, DH), L2W, L2b.reshape(1, DOUT))


def kernel(x, edge_index, edge_weight, batch,
           W1l, W1r, b1, W2l, W2r, b2, W3l, W3r, b3,
           L1W, L1b, L2W, L2b):
  del edge_weight
  pad = EP - E
  src = jnp.concatenate(
      [edge_index[0], jnp.zeros((pad,), jnp.int32)]).reshape(EP // CH, CH)
  dst = jnp.concatenate(
      [edge_index[1], jnp.full((pad,), NP - 1, jnp.int32)]).reshape(EP // CH, CH)
  zeros48 = jnp.zeros((NP, DH + 16), jnp.float32)
  zeros32 = jnp.zeros((NP, DH), jnp.float32)

  p1, q1 = _tc_project_in(x, W1l, W1r, b1)
  acc1 = _make_sc_aggregate(DH + 16)(p1, src, dst, zeros48)
  p2, q2, cnt = _tc_combine1(acc1[0], acc1[1], q1, W2l, W2r, b2)
  acc2 = _make_sc_aggregate(DH)(p2, src, dst, zeros32)
  p3, q3 = _tc_combine2(acc2[0], acc2[1], q2, cnt, W3l, W3r, b3)
  acc3 = _make_sc_aggregate(DH)(p3, src, dst, zeros32)
  out = _tc_head(acc3[0], acc3[1], q3, cnt, batch.reshape(N, 1),
                 L1W, L1b, L2W, L2b)
  return out


# speedup vs baseline: 20.8607x; 1.1771x over previous
import functools

import jax
import jax.numpy as jnp
from jax import lax
from jax.experimental import pallas as pl
from jax.experimental.pallas import tpu as pltpu
from jax.experimental.pallas import tpu_sc as plsc

N = 10000
E = 320000
DIN = 128
DH = 32
DOUT = 8
NG = 32

NC = 2
NS = 16
NW = NC * NS
CH = 128
EP = 327680
EPT = EP // NW
ITERS = EPT // CH
NP = 10240
RPT = NP // NS

BLK = 2000
GRID = N // BLK


@functools.lru_cache(maxsize=None)
def _make_sc_aggregate(W):
  mesh = plsc.VectorSubcoreMesh(core_axis_name="c", subcore_axis_name="s",
                                num_cores=NC, num_subcores=NS)

  @functools.partial(
      pl.kernel,
      out_type=jax.ShapeDtypeStruct((NC, NP, W), jnp.float32),
      mesh=mesh,
      scratch_types=[
          pltpu.VMEM((ITERS, CH), jnp.int32),
          pltpu.VMEM((ITERS, CH), jnp.int32),
          pltpu.VMEM((CH, W), jnp.float32),
          pltpu.VMEM((CH, W), jnp.float32),
          pltpu.VMEM((CH, W), jnp.float32),
          pltpu.VMEM((CH, W), jnp.float32),
          pltpu.VMEM_SHARED((N, W), jnp.float32),
          pltpu.VMEM_SHARED((NP, W), jnp.float32),
          pltpu.SemaphoreType.DMA,
          pltpu.SemaphoreType.DMA,
      ],
      compiler_params=pltpu.CompilerParams(use_tc_tiling_on_sc=False),
  )
  def agg(p_hbm, src_hbm, dst_hbm, zeros_hbm, out_hbm,
          sidx, didx, rows0, rows1, rows2, rows3, p_sh, acc, gsem, ssem):
    c = lax.axis_index("c")
    s = lax.axis_index("s")
    wid = c * NS + s
    rps = N // NS
    pltpu.sync_copy(p_hbm.at[pl.ds(s * rps, rps)],
                    p_sh.at[pl.ds(s * rps, rps)])
    pltpu.sync_copy(zeros_hbm.at[pl.ds(s * RPT, RPT)],
                    acc.at[pl.ds(s * RPT, RPT)])
    row0 = wid * ITERS
    pltpu.sync_copy(src_hbm.at[pl.ds(row0, ITERS)], sidx)
    pltpu.sync_copy(dst_hbm.at[pl.ds(row0, ITERS)], didx)
    plsc.subcore_barrier()

    def start_gather(j, buf):
      pltpu.async_copy(p_sh.at[sidx.at[j]], buf, gsem)

    def start_scatter(j, buf):
      pltpu.async_copy(buf, acc.at[didx.at[j]], ssem, add=True)

    def wait_gather(buf):
      pltpu.make_async_copy(p_sh.at[sidx.at[0]], buf, gsem).wait()

    def wait_scatter(buf):
      pltpu.make_async_copy(buf, acc.at[didx.at[0]], ssem).wait()

    bufs = (rows0, rows1, rows2, rows3)
    start_gather(0, rows0)
    start_gather(1, rows1)

    def body(j4, carry):
      j = 4 * j4
      for t in range(4):
        bt = bufs[t]
        bn = bufs[(t + 2) % 4]
        wait_gather(bt)
        if t < 2:
          @pl.when(j4 > 0)
          def _(bn=bn):
            wait_scatter(bn)
        else:
          wait_scatter(bn)
        if t < 2:
          start_gather(j + t + 2, bn)
        else:
          @pl.when(j4 < ITERS // 4 - 1)
          def _(bn=bn, jg=j + t + 2):
            start_gather(jg, bn)
        start_scatter(j + t, bt)
      return carry

    lax.fori_loop(0, ITERS // 4, body, 0)
    wait_scatter(rows2)
    wait_scatter(rows3)
    plsc.subcore_barrier()
    pltpu.sync_copy(acc.at[pl.ds(s * RPT, RPT)],
                    out_hbm.at[c, pl.ds(s * RPT, RPT)])

  return agg


def _k0_body(x_ref, wl_ref, wr_ref, b_ref, paug_ref, q_ref):
  x = x_ref[...]
  p = jnp.dot(x, wl_ref[...], preferred_element_type=jnp.float32)
  one = jnp.ones((BLK, 1), jnp.float32)
  pad = jnp.zeros((BLK, 15), jnp.float32)
  paug_ref[...] = jnp.concatenate([p, one, pad], axis=1)
  q_ref[...] = jnp.dot(x, wr_ref[...], preferred_element_type=jnp.float32) + b_ref[...]


def _tc_project_in(x, Wl, Wr, b):
  return pl.pallas_call(
      _k0_body,
      grid=(GRID,),
      in_specs=[
          pl.BlockSpec((BLK, DIN), lambda i: (i, 0)),
          pl.BlockSpec((DIN, DH), lambda i: (0, 0)),
          pl.BlockSpec((DIN, DH), lambda i: (0, 0)),
          pl.BlockSpec((1, DH), lambda i: (0, 0)),
      ],
      out_specs=[
          pl.BlockSpec((BLK, DH + 16), lambda i: (i, 0)),
          pl.BlockSpec((BLK, DH), lambda i: (i, 0)),
      ],
      out_shape=[
          jax.ShapeDtypeStruct((N, DH + 16), jnp.float32),
          jax.ShapeDtypeStruct((N, DH), jnp.float32),
      ],
  )(x, Wl, Wr, b.reshape(1, DH))


def _k1_body(acc_ref, q_ref, wl_ref, wr_ref, b_ref,
             p_ref, qn_ref, cnt_ref):
  a0 = acc_ref[0]
  a1 = acc_ref[1]
  cnt = a0[:, DH:DH + 1] + a1[:, DH:DH + 1]
  agg = a0[:, :DH] + a1[:, :DH]
  h = jnp.maximum(agg / jnp.maximum(cnt, 1.0) + q_ref[...], 0.0)
  p_ref[...] = jnp.dot(h, wl_ref[...], preferred_element_type=jnp.float32)
  qn_ref[...] = jnp.dot(h, wr_ref[...], preferred_element_type=jnp.float32) + b_ref[...]
  cnt_ref[...] = cnt


def _tc_combine1(acc, q, Wl, Wr, b):
  return pl.pallas_call(
      _k1_body,
      grid=(GRID,),
      in_specs=[
          pl.BlockSpec((NC, BLK, DH + 16), lambda i: (0, i, 0)),
          pl.BlockSpec((BLK, DH), lambda i: (i, 0)),
          pl.BlockSpec((DH, DH), lambda i: (0, 0)),
          pl.BlockSpec((DH, DH), lambda i: (0, 0)),
          pl.BlockSpec((1, DH), lambda i: (0, 0)),
      ],
      out_specs=[
          pl.BlockSpec((BLK, DH), lambda i: (i, 0)),
          pl.BlockSpec((BLK, DH), lambda i: (i, 0)),
          pl.BlockSpec((BLK, 1), lambda i: (i, 0)),
      ],
      out_shape=[
          jax.ShapeDtypeStruct((N, DH), jnp.float32),
          jax.ShapeDtypeStruct((N, DH), jnp.float32),
          jax.ShapeDtypeStruct((N, 1), jnp.float32),
      ],
  )(acc, q, Wl, Wr, b.reshape(1, DH))


def _k2_body(acc_ref, q_ref, cnt_ref, wl_ref, wr_ref, b_ref,
             p_ref, qn_ref):
  agg = acc_ref[0] + acc_ref[1]
  h = jnp.maximum(agg / jnp.maximum(cnt_ref[...], 1.0) + q_ref[...], 0.0)
  p_ref[...] = jnp.dot(h, wl_ref[...], preferred_element_type=jnp.float32)
  qn_ref[...] = jnp.dot(h, wr_ref[...], preferred_element_type=jnp.float32) + b_ref[...]


def _tc_combine2(acc, q, cnt, Wl, Wr, b):
  return pl.pallas_call(
      _k2_body,
      grid=(GRID,),
      in_specs=[
          pl.BlockSpec((NC, BLK, DH), lambda i: (0, i, 0)),
          pl.BlockSpec((BLK, DH), lambda i: (i, 0)),
          pl.BlockSpec((BLK, 1), lambda i: (i, 0)),
          pl.BlockSpec((DH, DH), lambda i: (0, 0)),
          pl.BlockSpec((DH, DH), lambda i: (0, 0)),
          pl.BlockSpec((1, DH), lambda i: (0, 0)),
      ],
      out_specs=[
          pl.BlockSpec((BLK, DH), lambda i: (i, 0)),
          pl.BlockSpec((BLK, DH), lambda i: (i, 0)),
      ],
      out_shape=[
          jax.ShapeDtypeStruct((N, DH), jnp.float32),
          jax.ShapeDtypeStruct((N, DH), jnp.float32),
      ],
  )(acc, q, cnt, Wl, Wr, b.reshape(1, DH))


def _k3_body(acc_ref, q_ref, cnt_ref, batch_ref,
             l1w_ref, l1b_ref, l2w_ref, l2b_ref, out_ref,
             gs_ref, gc_ref):
  i = pl.program_id(0)

  @pl.when(i == 0)
  def _():
    gs_ref[...] = jnp.zeros_like(gs_ref)
    gc_ref[...] = jnp.zeros_like(gc_ref)

  agg = acc_ref[0] + acc_ref[1]
  h = jnp.maximum(agg / jnp.maximum(cnt_ref[...], 1.0) + q_ref[...], 0.0)
  groups = lax.broadcasted_iota(jnp.int32, (BLK, NG), 1)
  onehot = (batch_ref[...] == groups).astype(jnp.float32)
  gs_ref[...] += lax.dot_general(
      onehot, h, (((0,), (0,)), ((), ())),
      preferred_element_type=jnp.float32)
  gc_ref[...] += lax.dot_general(
      onehot, jnp.ones((BLK, DH), jnp.float32), (((0,), (0,)), ((), ())),
      preferred_element_type=jnp.float32)

  @pl.when(i == GRID - 1)
  def _():
    g = gs_ref[...] / jnp.maximum(gc_ref[...], 1.0)
    z = jnp.maximum(
        jnp.dot(g, l1w_ref[...], preferred_element_type=jnp.float32)
        + l1b_ref[...], 0.0)
    out_ref[...] = jnp.dot(
        z, l2w_ref[...], preferred_element_type=jnp.float32) + l2b_ref[...]


def _tc_head(acc, q, cnt, batch2d, L1W, L1b, L2W, L2b):
  return pl.pallas_call(
      _k3_body,
      grid=(GRID,),
      in_specs=[
          pl.BlockSpec((NC, BLK, DH), lambda i: (0, i, 0)),
          pl.BlockSpec((BLK, DH), lambda i: (i, 0)),
          pl.BlockSpec((BLK, 1), lambda i: (i, 0)),
          pl.BlockSpec((BLK, 1), lambda i: (i, 0)),
          pl.BlockSpec((DH, DH), lambda i: (0, 0)),
          pl.BlockSpec((1, DH), lambda i: (0, 0)),
          pl.BlockSpec((DH, DOUT), lambda i: (0, 0)),
          pl.BlockSpec((1, DOUT), lambda i: (0, 0)),
      ],
      out_specs=pl.BlockSpec((NG, DOUT), lambda i: (0, 0)),
      out_shape=jax.ShapeDtypeStruct((NG, DOUT), jnp.float32),
      scratch_shapes=[
          pltpu.VMEM((NG, DH), jnp.float32),
          pltpu.VMEM((NG, DH), jnp.float32),
      ],
  )(acc, q, cnt, batch2d, L1W, L1b.reshape(1, DH), L2W, L2b.reshape(1, DOUT))


def kernel(x, edge_index, edge_weight, batch,
           W1l, W1r, b1, W2l, W2r, b2, W3l, W3r, b3,
           L1W, L1b, L2W, L2b):
  del edge_weight
  pad = EP - E
  src = jnp.concatenate(
      [edge_index[0], jnp.zeros((pad,), jnp.int32)]).reshape(EP // CH, CH)
  dst = jnp.concatenate(
      [edge_index[1], jnp.full((pad,), NP - 1, jnp.int32)]).reshape(EP // CH, CH)
  zeros48 = jnp.zeros((NP, DH + 16), jnp.float32)
  zeros32 = jnp.zeros((NP, DH), jnp.float32)

  p1, q1 = _tc_project_in(x, W1l, W1r, b1)
  acc1 = _make_sc_aggregate(DH + 16)(p1, src, dst, zeros48)
  p2, q2, cnt = _tc_combine1(acc1, q1, W2l, W2r, b2)
  acc2 = _make_sc_aggregate(DH)(p2, src, dst, zeros32)
  p3, q3 = _tc_combine2(acc2, q2, cnt, W3l, W3r, b3)
  acc3 = _make_sc_aggregate(DH)(p3, src, dst, zeros32)
  out = _tc_head(acc3, q3, cnt, batch.reshape(N, 1),
                 L1W, L1b, L2W, L2b)
  return out
